# baseline (speedup 1.0000x reference)
import jax
import jax.numpy as jnp
from jax.experimental import pallas as pl
from jax.experimental.pallas import tpu as pltpu

_IOU_T = 0.3
_SCORE_T = 0.5
_B = 512


def _nms_pull_kernel(cols_ref, rows_ref, valid_ref, dets_ref, kept_s, g_s):
    k = pl.program_id(0)
    B = cols_ref.shape[2]

    @pl.when(k == 0)
    def _init():
        kept_s[...] = jnp.zeros_like(kept_s)
        g_s[0] = 0

    cc = cols_ref[k]
    x1c = cc[0:1]
    y1c = cc[1:2]
    x2c = cc[2:3]
    y2c = cc[3:4]
    ac = jnp.maximum(x2c - x1c, 0.0) * jnp.maximum(y2c - y1c, 0.0)

    def iou_gt_rows(rows):
        x1r = rows[:, 0:1]
        y1r = rows[:, 1:2]
        x2r = rows[:, 2:3]
        y2r = rows[:, 3:4]
        ar = jnp.maximum(x2r - x1r, 0.0) * jnp.maximum(y2r - y1r, 0.0)
        xx1 = jnp.maximum(x1r, x1c)
        yy1 = jnp.maximum(y1r, y1c)
        xx2 = jnp.minimum(x2r, x2c)
        yy2 = jnp.minimum(y2r, y2c)
        inter = jnp.maximum(xx2 - xx1, 0.0) * jnp.maximum(yy2 - yy1, 0.0)
        iou = inter / (ar + ac - inter + 1e-9)
        return (iou > _IOU_T).astype(jnp.float32)

    def mm(a, b):
        return jax.lax.dot_general(
            a, b, (((1,), (0,)), ((), ())),
            preferred_element_type=jnp.float32,
            precision=jax.lax.Precision.HIGHEST,
        )

    ones_row = jnp.ones((1, B), jnp.float32)

    @pl.when(jnp.any(valid_ref[k] > 0.0))
    def _process():
        g = g_s[0]
        nch = (g + B - 1) // B

        def chunk_body(j, sup):
            return sup + mm(ones_row, iou_gt_rows(kept_s[j]))

        sup = jax.lax.fori_loop(0, nch, chunk_body,
                                jnp.zeros((1, B), jnp.float32))
        alive = jnp.where(sup > 0.0, 0.0, valid_ref[k])

        M = iou_gt_rows(rows_ref[k])
        rix = jax.lax.broadcasted_iota(jnp.int32, (B, B), 0)
        cix = jax.lax.broadcasted_iota(jnp.int32, (B, B), 1)
        M = jnp.where(rix < cix, M, 0.0)

        def cond(carry):
            return carry[1]

        def body(carry):
            K, _ = carry
            newK = jnp.where(mm(K, M) > 0.0, 0.0, alive)
            return newK, jnp.any(newK != K)

        K, _ = jax.lax.while_loop(cond, body, (alive, jnp.bool_(True)))

        dets_ref[0] = cc * K

        off = g - (g // B) * B
        cw = g // B
        lt = jnp.where(rix <= cix, 1.0, 0.0)
        psum = mm(K, lt)
        t = off + psum.astype(jnp.int32) - 1
        ci = jax.lax.broadcasted_iota(jnp.int32, (B, B), 0)
        sel1 = jnp.where((ci == t) & (K > 0.0), 1.0, 0.0)
        sel2 = jnp.where((ci + B == t) & (K > 0.0), 1.0, 0.0)
        br = rows_ref[k]
        kept_s[cw] = kept_s[cw] + mm(sel1, br)
        kept_s[cw + 1] = kept_s[cw + 1] + mm(sel2, br)
        g_s[0] = g + jnp.sum(K).astype(jnp.int32)

    @pl.when(jnp.logical_not(jnp.any(valid_ref[k] > 0.0)))
    def _dead():
        dets_ref[0] = jnp.zeros_like(dets_ref[0])


def kernel(boxes, scores):
    N = boxes.shape[0]
    nblk = (N + _B - 1) // _B
    Np = nblk * _B
    pad = Np - N

    order = jnp.argsort(-scores)
    b = jnp.take(boxes, order, axis=0)
    s = jnp.take(scores, order, axis=0)
    valid = (s > _SCORE_T).astype(jnp.float32)

    bp = jnp.pad(b, ((0, pad), (0, 0)))
    sp = jnp.pad(s, ((0, pad),))
    vp = jnp.pad(valid, ((0, pad),))

    cols = jnp.transpose(
        jnp.reshape(jnp.concatenate([bp, sp[:, None]], axis=1), (nblk, _B, 5)),
        (0, 2, 1))
    rows = jnp.reshape(bp, (nblk, _B, 4))
    v3 = jnp.reshape(vp, (nblk, 1, _B))

    out = pl.pallas_call(
        _nms_pull_kernel,
        grid=(nblk,),
        in_specs=[
            pl.BlockSpec((nblk, 5, _B), lambda k: (0, 0, 0)),
            pl.BlockSpec((nblk, _B, 4), lambda k: (0, 0, 0)),
            pl.BlockSpec((nblk, 1, _B), lambda k: (0, 0, 0)),
        ],
        out_specs=pl.BlockSpec((1, 5, _B), lambda k: (k, 0, 0)),
        out_shape=jax.ShapeDtypeStruct((nblk, 5, _B), jnp.float32),
        scratch_shapes=[
            pltpu.VMEM((nblk + 1, _B, 4), jnp.float32),
            pltpu.SMEM((1,), jnp.int32),
        ],
        compiler_params=pltpu.CompilerParams(
            dimension_semantics=("arbitrary",)),
    )(cols, rows, v3)

    dets_sorted = jnp.reshape(jnp.transpose(out, (0, 2, 1)), (Np, 5))[:N]
    return jnp.zeros((N, 5), boxes.dtype).at[order].set(dets_sorted)


# speedup vs baseline: 1.4815x; 1.4815x over previous
import jax
import jax.numpy as jnp
from jax.experimental import pallas as pl
from jax.experimental.pallas import tpu as pltpu

_IOU_T = 0.3
_SCORE_T = 0.5
_B = 512


def _nms_pull_kernel(cols_ref, rows_ref, valid_ref, keep_ref, kept_s, g_s):
    k = pl.program_id(0)
    B = cols_ref.shape[2]

    @pl.when(k == 0)
    def _init():
        kept_s[...] = jnp.zeros_like(kept_s)
        g_s[0] = 0

    cc = cols_ref[k]
    x1c = cc[0:1]
    y1c = cc[1:2]
    x2c = cc[2:3]
    y2c = cc[3:4]
    ac = jnp.maximum(x2c - x1c, 0.0) * jnp.maximum(y2c - y1c, 0.0)

    def iou_gt_rows(rows):
        x1r = rows[:, 0:1]
        y1r = rows[:, 1:2]
        x2r = rows[:, 2:3]
        y2r = rows[:, 3:4]
        ar = jnp.maximum(x2r - x1r, 0.0) * jnp.maximum(y2r - y1r, 0.0)
        xx1 = jnp.maximum(x1r, x1c)
        yy1 = jnp.maximum(y1r, y1c)
        xx2 = jnp.minimum(x2r, x2c)
        yy2 = jnp.minimum(y2r, y2c)
        inter = jnp.maximum(xx2 - xx1, 0.0) * jnp.maximum(yy2 - yy1, 0.0)
        iou = inter / (ar + ac - inter + 1e-9)
        return (iou > _IOU_T).astype(jnp.float32)

    def mm(a, b):
        return jax.lax.dot_general(
            a, b, (((1,), (0,)), ((), ())),
            preferred_element_type=jnp.float32,
            precision=jax.lax.Precision.HIGHEST,
        )

    ones_row = jnp.ones((1, B), jnp.float32)

    @pl.when(jnp.any(valid_ref[k] > 0.0))
    def _process():
        g = g_s[0]
        nch = (g + B - 1) // B

        def chunk_body(j, sup):
            return sup + mm(ones_row, iou_gt_rows(kept_s[j]))

        sup = jax.lax.fori_loop(0, nch, chunk_body,
                                jnp.zeros((1, B), jnp.float32))
        alive = jnp.where(sup > 0.0, 0.0, valid_ref[k])

        M = iou_gt_rows(rows_ref[k])
        rix = jax.lax.broadcasted_iota(jnp.int32, (B, B), 0)
        cix = jax.lax.broadcasted_iota(jnp.int32, (B, B), 1)
        M = jnp.where(rix < cix, M, 0.0)

        def cond(carry):
            return carry[1]

        def body(carry):
            K, _ = carry
            newK = jnp.where(mm(K, M) > 0.0, 0.0, alive)
            return newK, jnp.any(newK != K)

        K, _ = jax.lax.while_loop(cond, body, (alive, jnp.bool_(True)))

        keep_ref[0] = K

        off = g - (g // B) * B
        cw = g // B
        lt = jnp.where(rix <= cix, 1.0, 0.0)
        psum = mm(K, lt)
        t = off + psum.astype(jnp.int32) - 1
        ci = jax.lax.broadcasted_iota(jnp.int32, (B, B), 0)
        sel1 = jnp.where((ci == t) & (K > 0.0), 1.0, 0.0)
        sel2 = jnp.where((ci + B == t) & (K > 0.0), 1.0, 0.0)
        br = rows_ref[k]
        kept_s[cw] = kept_s[cw] + mm(sel1, br)
        kept_s[cw + 1] = kept_s[cw + 1] + mm(sel2, br)
        g_s[0] = g + jnp.sum(K).astype(jnp.int32)

    @pl.when(jnp.logical_not(jnp.any(valid_ref[k] > 0.0)))
    def _dead():
        keep_ref[0] = jnp.zeros_like(keep_ref[0])


def kernel(boxes, scores):
    N = boxes.shape[0]
    nblk = (N + _B - 1) // _B
    Np = nblk * _B
    pad = Np - N

    iota = jnp.arange(N, dtype=jnp.int32)
    negs, order, x1, y1, x2, y2 = jax.lax.sort(
        (-scores, iota, boxes[:, 0], boxes[:, 1], boxes[:, 2], boxes[:, 3]),
        dimension=0, is_stable=True, num_keys=1)
    svalid = (negs < -_SCORE_T).astype(jnp.float32)

    def padv(v):
        return jnp.pad(v, ((0, pad),))

    x1p, y1p, x2p, y2p = (padv(v) for v in (x1, y1, x2, y2))
    vp = padv(svalid)

    cols = jnp.stack([jnp.reshape(v, (nblk, _B)) for v in (x1p, y1p, x2p, y2p)],
                     axis=1)
    rows = jnp.stack([jnp.reshape(v, (nblk, _B)) for v in (x1p, y1p, x2p, y2p)],
                     axis=2)
    v3 = jnp.reshape(vp, (nblk, 1, _B))

    keep = pl.pallas_call(
        _nms_pull_kernel,
        grid=(nblk,),
        in_specs=[
            pl.BlockSpec((nblk, 4, _B), lambda k: (0, 0, 0)),
            pl.BlockSpec((nblk, _B, 4), lambda k: (0, 0, 0)),
            pl.BlockSpec((nblk, 1, _B), lambda k: (0, 0, 0)),
        ],
        out_specs=pl.BlockSpec((1, 1, _B), lambda k: (k, 0, 0)),
        out_shape=jax.ShapeDtypeStruct((nblk, 1, _B), jnp.float32),
        scratch_shapes=[
            pltpu.VMEM((nblk + 1, _B, 4), jnp.float32),
            pltpu.SMEM((1,), jnp.int32),
        ],
        compiler_params=pltpu.CompilerParams(
            dimension_semantics=("arbitrary",)),
    )(cols, rows, v3)

    keep_sorted = jnp.reshape(keep, (Np,))[:N]
    _, keep_orig = jax.lax.sort((order, keep_sorted), dimension=0,
                                is_stable=False, num_keys=1)
    dets = jnp.concatenate([boxes, scores[:, None]], axis=1)
    return dets * keep_orig[:, None]
